# Initial kernel scaffold; baseline (speedup 1.0000x reference)
import jax
import jax.numpy as jnp
from jax.experimental import pallas as pl


def kernel(index, scores):
    raise NotImplementedError("write your pallas kernel here")


# speedup vs baseline: 2.6163x; 2.6163x over previous
import jax
import jax.numpy as jnp
from jax.experimental import pallas as pl
from jax.experimental.pallas import tpu as pltpu

_B, _H, _U, _LK = 4, 16, 64, 4096
_BH = _B * _H


def _mask_kernel(idx_ref, out_ref):
    idx = idx_ref[...].reshape(1, _U, 1)
    cols = jax.lax.broadcasted_iota(jnp.int32, (1, _U, _LK), 2)
    out_ref[...] = cols > idx


def kernel(index, scores):
    del scores
    idx3 = index.reshape(_BH, 1, _U)
    out = pl.pallas_call(
        _mask_kernel,
        grid=(_BH,),
        in_specs=[pl.BlockSpec((1, 1, _U), lambda i: (i, 0, 0))],
        out_specs=pl.BlockSpec((1, _U, _LK), lambda i: (i, 0, 0)),
        out_shape=jax.ShapeDtypeStruct((_BH, _U, _LK), jnp.bool_),
    )(idx3)
    return out.reshape(_B, _H, _U, _LK)
